# baseline (speedup 1.0000x reference)
import functools

import jax
import jax.numpy as jnp
from jax import lax
from jax.experimental import pallas as pl
from jax.experimental.pallas import tpu as pltpu
from jax.experimental.pallas import tpu_sc as plsc

_VOCAB = 100000
_D = 64
_NTOK = 5
_BATCH = 4096
_SEQ = 200
_SEQ_E = _SEQ - _NTOK
_NW = 32
_BC = _BATCH // _NW
_DT = _D // 8
_L = 16


@functools.partial(
    pl.kernel,
    out_type=jax.ShapeDtypeStruct((_SEQ, _DT, _NW, 8, _BC), jnp.float32),
    mesh=plsc.VectorSubcoreMesh(core_axis_name="c", subcore_axis_name="s"),
    compiler_params=pltpu.CompilerParams(
        use_tc_tiling_on_sc=False, needs_layout_passes=False),
    scratch_types=[
        pltpu.VMEM((_SEQ, _BC), jnp.int32),
        pltpu.VMEM((_BC, _D), jnp.float32),
        pltpu.VMEM((_BC, _D), jnp.float32),
        pltpu.VMEM((_BC, _D), jnp.float32),
        pltpu.VMEM((_BC, _D), jnp.float32),
        pltpu.VMEM((_DT, 8, _BC + 8), jnp.float32),
        pltpu.VMEM((_DT, 8, _BC + 8), jnp.float32),
        pltpu.VMEM((_NTOK, _DT, 8, _BC), jnp.float32),
        pltpu.VMEM((_NTOK * _D, _L), jnp.float32),
        pltpu.VMEM((_NTOK * _D, _L), jnp.float32),
        pltpu.VMEM((_D // _L, _L), jnp.int32),
        pltpu.VMEM((1, _L), jnp.int32),
        pltpu.VMEM((_BC, _L), jnp.int32),
        pltpu.SemaphoreType.DMA,
        pltpu.SemaphoreType.DMA,
        pltpu.SemaphoreType.DMA,
        pltpu.SemaphoreType.DMA,
        pltpu.SemaphoreType.DMA,
        pltpu.SemaphoreType.DMA,
        pltpu.SemaphoreType.DMA,
    ],
)
def _sc_embed(idxt_hbm, table_hbm, var_hbm, avg_hbm, smp_hbm,
              dtv_hbm, drv_hbm, bspl_hbm, out_hbm,
              idx_v, rows0, rows1, rows2, rows3, pl0, pl1, smp_v, var_v,
              avg_v, dtv_v, drv_v, bspl_v, g0, g1, g2, g3, w0, w1, psem):
    rows = (rows0, rows1, rows2, rows3)
    planes = (pl0, pl1)
    gsem = (g0, g1, g2, g3)
    wsem = (w0, w1)
    nc = 2
    wid = lax.axis_index("s") * nc + lax.axis_index("c")
    c0 = wid * _BC

    pltpu.sync_copy(idxt_hbm.at[pl.ds(0, _SEQ), pl.ds(c0, _BC)], idx_v)
    pltpu.sync_copy(var_hbm, var_v)
    pltpu.sync_copy(avg_hbm, avg_v)
    pltpu.sync_copy(dtv_hbm, dtv_v)
    pltpu.sync_copy(drv_hbm, drv_v)
    pltpu.sync_copy(bspl_hbm, bspl_v)

    def issue_gather(s, p):
        pltpu.make_async_copy(
            table_hbm.at[idx_v.at[s]], rows[p], gsem[p]).start()

    def drain_gather(p):
        pltpu.make_async_copy(
            table_hbm.at[pl.ds(0, _BC)], rows[p], gsem[p]).wait()

    def issue_write(s, p):
        pltpu.make_async_copy(
            planes[p].at[pl.ds(0, _DT), pl.ds(0, 8), pl.ds(0, _BC)],
            out_hbm.at[s, pl.ds(0, _DT), wid], wsem[p]).start()

    def drain_write(p):
        pltpu.make_async_copy(
            planes[p].at[pl.ds(0, _DT), pl.ds(0, 8), pl.ds(0, _BC)],
            out_hbm.at[0, pl.ds(0, _DT), wid], wsem[p]).wait()

    for k in range(4):
        issue_gather(k, k)

    pltpu.sync_copy(
        smp_hbm.at[pl.ds(0, _NTOK), pl.ds(0, _DT), pl.ds(0, 8),
                   pl.ds(c0, _BC)], smp_v)

    def pref_dt(t, carry):
        for r in range(_NTOK):
            for dr in range(8):
                row = r * _D + t * 8 + dr
                vv = var_v[row, pl.ds(0, _L)]
                av = avg_v[row, pl.ds(0, _L)]
                for g in range(_BC // _L):
                    sl = pl.ds(g * _L, _L)
                    smp_v[r, t, dr, sl] = smp_v[r, t, dr, sl] * vv + av
        return carry

    lax.fori_loop(0, _DT, pref_dt, 0)
    for r in range(_NTOK):
        pltpu.make_async_copy(
            smp_v.at[r], out_hbm.at[_SEQ_E + r, pl.ds(0, _DT), wid],
            psem).start()

    _UNR = 16

    def transpose_unit(rk, p):
        def b_body(j, carry):
            b0 = j * _UNR
            dr = drv_v[0, pl.ds(0, _L)]
            dts = [dtv_v[c, pl.ds(0, _L)] for c in range(_D // _L)]
            for k in range(_UNR):
                bv = bspl_v[b0 + k, pl.ds(0, _L)]
                for c in range(_D // _L):
                    v = rows[rk][b0 + k, pl.ds(c * _L, _L)]
                    plsc.store_scatter(planes[p], [dts[c], dr, bv], v)
            return carry

        lax.fori_loop(0, _BC // _UNR, b_body, 0)

    def quad(j, carry):
        for k in range(4):
            s = 4 * j + k
            p = k % 2
            drain_gather(k)

            @pl.when(s >= 2)
            def _():
                drain_write(p)

            @pl.when(s < _SEQ_E)
            def _():
                transpose_unit(k, p)
                issue_write(s, p)

            @pl.when(s + 4 < _SEQ_E + 1)
            def _():
                issue_gather(s + 4, k)

        return carry

    lax.fori_loop(0, (_SEQ_E + 1) // 4, quad, 0)
    drain_write(0)
    for r in range(_NTOK):
        pltpu.make_async_copy(
            smp_v.at[0], out_hbm.at[_SEQ_E, pl.ds(0, _DT), wid], psem).wait()


def kernel(tokens, table, avg, var):
    idxt = jnp.roll(tokens, -_NTOK, axis=1).T
    sample = jax.random.normal(
        jax.random.key(1), (_BATCH, _NTOK, _D), dtype=jnp.float32)
    smpt = jnp.transpose(sample, (1, 2, 0)).reshape(_NTOK, _DT, 8, _BATCH)
    vart = jnp.tile(var.reshape(-1, 1), (1, _L))
    avgt = jnp.tile(avg.reshape(-1, 1), (1, _L))
    lane = jnp.arange(_L, dtype=jnp.int32)
    dl = jnp.arange(_D, dtype=jnp.int32).reshape(_D // _L, _L)
    dtv = dl // 8
    drv = (lane % 8).reshape(1, _L)
    bspl = jnp.tile(
        jnp.arange(_BC, dtype=jnp.int32).reshape(-1, 1), (1, _L))
    out5 = _sc_embed(idxt, table, vart, avgt, smpt, dtv, drv, bspl)
    return jnp.reshape(jnp.transpose(out5, (2, 4, 0, 1, 3)),
                       (_BATCH, _SEQ, _D))


# speedup vs baseline: 2.2509x; 2.2509x over previous
import functools

import jax
import jax.numpy as jnp
from jax import lax
from jax.experimental import pallas as pl
from jax.experimental.pallas import tpu as pltpu
from jax.experimental.pallas import tpu_sc as plsc

_VOCAB = 100000
_D = 64
_NTOK = 5
_BATCH = 4096
_SEQ = 200
_SEQ_E = _SEQ - _NTOK
_NW = 32
_BC = _BATCH // _NW
_DT = _D // 8
_L = 16


@functools.partial(
    pl.kernel,
    out_type=jax.ShapeDtypeStruct((_SEQ, _DT, _NW, 8, _BC), jnp.float32),
    mesh=plsc.VectorSubcoreMesh(core_axis_name="c", subcore_axis_name="s"),
    compiler_params=pltpu.CompilerParams(
        use_tc_tiling_on_sc=False, needs_layout_passes=False),
    scratch_types=[
        pltpu.VMEM((_SEQ, _BC), jnp.int32),
        pltpu.VMEM((_BC, _D), jnp.float32),
        pltpu.VMEM((_BC, _D), jnp.float32),
        pltpu.VMEM((_BC, _D), jnp.float32),
        pltpu.VMEM((_BC, _D), jnp.float32),
        pltpu.VMEM((_DT, 8, _BC + 8), jnp.float32),
        pltpu.VMEM((_DT, 8, _BC + 8), jnp.float32),
        pltpu.VMEM((_NTOK, _DT, 8, _BC), jnp.float32),
        pltpu.VMEM((_NTOK * _D, _L), jnp.float32),
        pltpu.VMEM((_NTOK * _D, _L), jnp.float32),
        pltpu.VMEM((_D // _L, _L), jnp.int32),
        pltpu.VMEM((1, _L), jnp.int32),
        pltpu.VMEM((_BC, _L), jnp.int32),
        pltpu.SemaphoreType.DMA,
        pltpu.SemaphoreType.DMA,
        pltpu.SemaphoreType.DMA,
        pltpu.SemaphoreType.DMA,
        pltpu.SemaphoreType.DMA,
        pltpu.SemaphoreType.DMA,
        pltpu.SemaphoreType.DMA,
    ],
)
def _sc_embed(idxt_hbm, table_hbm, var_hbm, avg_hbm, smp_hbm,
              dtv_hbm, drv_hbm, bspl_hbm, out_hbm,
              idx_v, rows0, rows1, rows2, rows3, pl0, pl1, smp_v, var_v,
              avg_v, dtv_v, drv_v, bspl_v, g0, g1, g2, g3, w0, w1, psem):
    rows = (rows0, rows1, rows2, rows3)
    planes = (pl0, pl1)
    gsem = (g0, g1, g2, g3)
    wsem = (w0, w1)
    nc = 2
    wid = lax.axis_index("s") * nc + lax.axis_index("c")
    c0 = wid * _BC

    pltpu.sync_copy(idxt_hbm.at[pl.ds(0, _SEQ), pl.ds(c0, _BC)], idx_v)
    pltpu.sync_copy(var_hbm, var_v)
    pltpu.sync_copy(avg_hbm, avg_v)
    pltpu.sync_copy(dtv_hbm, dtv_v)
    pltpu.sync_copy(drv_hbm, drv_v)
    pltpu.sync_copy(bspl_hbm, bspl_v)

    def issue_gather(s, p):
        pltpu.make_async_copy(
            table_hbm.at[idx_v.at[s]], rows[p], gsem[p]).start()

    def drain_gather(p):
        pltpu.make_async_copy(
            table_hbm.at[pl.ds(0, _BC)], rows[p], gsem[p]).wait()

    def issue_write(s, p):
        pltpu.make_async_copy(
            planes[p].at[pl.ds(0, _DT), pl.ds(0, 8), pl.ds(0, _BC)],
            out_hbm.at[s, pl.ds(0, _DT), wid], wsem[p]).start()

    def drain_write(p):
        pltpu.make_async_copy(
            planes[p].at[pl.ds(0, _DT), pl.ds(0, 8), pl.ds(0, _BC)],
            out_hbm.at[0, pl.ds(0, _DT), wid], wsem[p]).wait()

    for k in range(4):
        issue_gather(k, k)

    pltpu.sync_copy(
        smp_hbm.at[pl.ds(0, _NTOK), pl.ds(0, _DT), pl.ds(0, 8),
                   pl.ds(c0, _BC)], smp_v)

    def pref_dt(t, carry):
        for r in range(_NTOK):
            for dr in range(8):
                row = r * _D + t * 8 + dr
                vv = var_v[row, pl.ds(0, _L)]
                av = avg_v[row, pl.ds(0, _L)]
                for g in range(_BC // _L):
                    sl = pl.ds(g * _L, _L)
                    smp_v[r, t, dr, sl] = smp_v[r, t, dr, sl] * vv + av
        return carry

    lax.fori_loop(0, _DT, pref_dt, 0)
    for r in range(_NTOK):
        pltpu.make_async_copy(
            smp_v.at[r], out_hbm.at[_SEQ_E + r, pl.ds(0, _DT), wid],
            psem).start()

    _UNR = 16

    def transpose_unit(rk, p):
        @plsc.parallel_loop(0, _BC, unroll=_UNR)
        def b_body(b):
            dr = drv_v[0, pl.ds(0, _L)]
            bv = bspl_v[b, pl.ds(0, _L)]
            for c in range(_D // _L):
                v = rows[rk][b, pl.ds(c * _L, _L)]
                plsc.store_scatter(planes[p], [dtv_v[c, pl.ds(0, _L)], dr, bv], v)

    def quad(j, carry):
        for k in range(4):
            s = 4 * j + k
            p = k % 2
            drain_gather(k)

            @pl.when(s >= 2)
            def _():
                drain_write(p)

            @pl.when(s < _SEQ_E)
            def _():
                transpose_unit(k, p)
                issue_write(s, p)

            @pl.when(s + 4 < _SEQ_E + 1)
            def _():
                issue_gather(s + 4, k)

        return carry

    lax.fori_loop(0, (_SEQ_E + 1) // 4, quad, 0)
    drain_write(0)
    for r in range(_NTOK):
        pltpu.make_async_copy(
            smp_v.at[0], out_hbm.at[_SEQ_E, pl.ds(0, _DT), wid], psem).wait()


def kernel(tokens, table, avg, var):
    idxt = jnp.roll(tokens, -_NTOK, axis=1).T
    sample = jax.random.normal(
        jax.random.key(1), (_BATCH, _NTOK, _D), dtype=jnp.float32)
    smpt = jnp.transpose(sample, (1, 2, 0)).reshape(_NTOK, _DT, 8, _BATCH)
    vart = jnp.tile(var.reshape(-1, 1), (1, _L))
    avgt = jnp.tile(avg.reshape(-1, 1), (1, _L))
    lane = jnp.arange(_L, dtype=jnp.int32)
    dl = jnp.arange(_D, dtype=jnp.int32).reshape(_D // _L, _L)
    dtv = dl // 8
    drv = (lane % 8).reshape(1, _L)
    bspl = jnp.tile(
        jnp.arange(_BC, dtype=jnp.int32).reshape(-1, 1), (1, _L))
    out5 = _sc_embed(idxt, table, vart, avgt, smpt, dtv, drv, bspl)
    return jnp.reshape(jnp.transpose(out5, (2, 4, 0, 1, 3)),
                       (_BATCH, _SEQ, _D))
